# baseline (speedup 1.0000x reference)
import jax
import jax.numpy as jnp
from jax.experimental import pallas as pl

EPS = 1e-5


def _final_body(agg_ref, x_ref, o_ref):
    agg = agg_ref[...]
    agg = jnp.where(jnp.isneginf(agg), 0.0, agg)
    s = agg + x_ref[...]
    o_ref[...] = jnp.where(s >= 0, s, 0.01 * s)


def _bn(h, g, b):
    mu = jnp.mean(h, axis=0, keepdims=True)
    var = jnp.var(h, axis=0, keepdims=True)
    return (h - mu) / jnp.sqrt(var + EPS) * g + b


def _lrelu(h):
    return jnp.where(h >= 0, h, 0.01 * h)


def kernel(x, edge_index, W1, b1, g1, be1, W2, b2, g2, be2, W3, b3):
    src = edge_index[0]
    dst = edge_index[1]
    x_i = jnp.take(x, dst, axis=0)
    x_j = jnp.take(x, src, axis=0)
    m = jnp.concatenate([x_i, x_j - x_i], axis=-1)
    h = _lrelu(_bn(m @ W1.T + b1, g1, be1))
    h = _lrelu(_bn(h @ W2.T + b2, g2, be2))
    h = h @ W3.T + b3
    agg = jax.ops.segment_max(h, dst, num_segments=x.shape[0])
    N, D = x.shape
    out = pl.pallas_call(
        _final_body,
        out_shape=jax.ShapeDtypeStruct((N, D), jnp.float32),
        grid=(N // 1000,),
        in_specs=[
            pl.BlockSpec((1000, D), lambda i: (i, 0)),
            pl.BlockSpec((1000, D), lambda i: (i, 0)),
        ],
        out_specs=pl.BlockSpec((1000, D), lambda i: (i, 0)),
    )(agg, x)
    return out


# speedup vs baseline: 1.3325x; 1.3325x over previous
import functools

import jax
import jax.numpy as jnp
from jax import lax
from jax.experimental import pallas as pl
from jax.experimental.pallas import tpu as pltpu
from jax.experimental.pallas import tpu_sc as plsc

EPS = 1e-5
N = 10000
E = 320000
D = 128
NW = 32
EPW = E // NW
GC = 400
NPW = 320
NPAD = NW * NPW
BP = 1280
TP = E // BP
PC = 400
SB = 80
NSB = PC // SB
FC = 256
BT = 1280
T = E // BT


def _uv_body(x_ref, wa_ref, wb_ref, u_ref, v_ref):
    xb = x_ref[...]
    u_ref[...] = jnp.dot(xb, wa_ref[...], preferred_element_type=jnp.float32)
    v_ref[...] = jnp.dot(xb, wb_ref[...], preferred_element_type=jnp.float32)


def _k1(x, wa, wb):
    return pl.pallas_call(
        _uv_body,
        out_shape=(jax.ShapeDtypeStruct((N, D), jnp.float32),
                   jax.ShapeDtypeStruct((N, D), jnp.float32)),
        grid=(10,),
        in_specs=[
            pl.BlockSpec((N // 10, D), lambda i: (i, 0)),
            pl.BlockSpec((D, D), lambda i: (0, 0)),
            pl.BlockSpec((D, D), lambda i: (0, 0)),
        ],
        out_specs=(pl.BlockSpec((N // 10, D), lambda i: (i, 0)),
                   pl.BlockSpec((N // 10, D), lambda i: (i, 0))),
    )(x, wa, wb)


def _pos_body(dst_ref, pos_ref, cnt_ref, acc_ref, l_ref):
    p = pl.program_id(0)
    t = pl.program_id(1)

    @pl.when(jnp.logical_and(p == 0, t == 0))
    def _init():
        acc_ref[...] = jnp.zeros_like(acc_ref)
        ri = lax.broadcasted_iota(jnp.int32, (BP, BP), 0)
        ci = lax.broadcasted_iota(jnp.int32, (BP, BP), 1)
        l_ref[...] = jnp.where(ri > ci, 1.0, 0.0)

    d = dst_ref[0]
    bkt = d.reshape(BP, 1) // NPW
    bi = lax.broadcasted_iota(jnp.int32, (BP, NW), 1)
    oh = jnp.where(bkt == bi, 1.0, 0.0)

    @pl.when(p == 0)
    def _count():
        acc_ref[0:1] += jnp.sum(oh, axis=0, keepdims=True)

    @pl.when(jnp.logical_and(p == 1, t == 0))
    def _mkstarts():
        cnt_ref[...] = acc_ref[0:1]
        ri = lax.broadcasted_iota(jnp.int32, (NW, NW), 0)
        ci = lax.broadcasted_iota(jnp.int32, (NW, NW), 1)
        m = jnp.where(ri < ci, 1.0, 0.0)
        acc_ref[2:3] = jnp.dot(acc_ref[0:1], m,
                               preferred_element_type=jnp.float32)
        acc_ref[0:1] = jnp.zeros((1, NW), jnp.float32)

    @pl.when(p == 1)
    def _emitpos():
        pw = jnp.dot(l_ref[...], oh, preferred_element_type=jnp.float32)
        base = acc_ref[0:1] + acc_ref[2:3]
        pp = jnp.sum(oh * (pw + base), axis=1)
        pos_ref[0] = pp.astype(jnp.int32).reshape(1, BP)
        acc_ref[0:1] += jnp.sum(oh, axis=0, keepdims=True)


def _kpos(dst3):
    pos3, cnt = pl.pallas_call(
        _pos_body,
        out_shape=(jax.ShapeDtypeStruct((TP, 1, BP), jnp.int32),
                   jax.ShapeDtypeStruct((1, NW), jnp.float32)),
        grid=(2, TP),
        in_specs=[pl.BlockSpec((1, 1, BP), lambda p, t: (t, 0, 0))],
        out_specs=(pl.BlockSpec((1, 1, BP),
                                lambda p, t: (jnp.where(p == 1, t, 0), 0, 0)),
                   pl.BlockSpec((1, NW), lambda p, t: (0, 0))),
        scratch_shapes=[pltpu.VMEM((8, NW), jnp.float32),
                        pltpu.VMEM((BP, BP), jnp.float32)],
    )(dst3)
    return pos3, cnt


def _kperm(pos, dst):
    mesh = plsc.VectorSubcoreMesh(core_axis_name="c", subcore_axis_name="s")

    @functools.partial(
        pl.kernel,
        out_type=(jax.ShapeDtypeStruct((E + FC,), jnp.int32),
                  jax.ShapeDtypeStruct((E + FC,), jnp.int32)),
        mesh=mesh,
        scratch_types=[
            pltpu.VMEM((NSB, SB), jnp.int32),
            pltpu.VMEM((NSB, SB), jnp.int32),
            pltpu.VMEM((NSB, SB), jnp.int32),
            pltpu.VMEM((FC,), jnp.int32),
            pltpu.SemaphoreType.DMA,
            pltpu.SemaphoreType.DMA,
        ],
    )
    def k(pos_hbm, dst_hbm, ids_hbm, dsts_hbm, posb, db, idb, zb, sem1, sem2):
        c = lax.axis_index("c")
        s = lax.axis_index("s")
        wid = s * 2 + c
        base = wid * EPW
        iota = lax.iota(jnp.int32, 16)

        def chunk(j, _):
            e0 = base + j * PC
            for k_ in range(NSB):
                pltpu.sync_copy(pos_hbm.at[pl.ds(e0 + k_ * SB, SB)],
                                posb.at[k_])
                pltpu.sync_copy(dst_hbm.at[pl.ds(e0 + k_ * SB, SB)],
                                db.at[k_])

            def mk_ids(i, _):
                for k_ in range(NSB):
                    sl = pl.ds(pl.multiple_of(i * 16, 16), 16)
                    idb[k_, sl] = (jnp.full((16,), e0 + k_ * SB, jnp.int32)
                                   + i * 16 + iota)
                return 0
            lax.fori_loop(0, SB // 16, mk_ids, 0)
            for k_ in range(NSB):
                cp1 = pltpu.async_copy(idb.at[k_], ids_hbm.at[posb.at[k_]],
                                       sem1)
                cp2 = pltpu.async_copy(db.at[k_], dsts_hbm.at[posb.at[k_]],
                                       sem2)
                cp1.wait()
                cp2.wait()
            return 0
        lax.fori_loop(0, EPW // PC, chunk, 0)

        @pl.when(wid == NW - 1)
        def _pad():
            def z(i, _):
                zb[pl.ds(pl.multiple_of(i * 16, 16), 16)] = jnp.zeros(
                    (16,), jnp.int32)
                return 0
            lax.fori_loop(0, FC // 16, z, 0)
            pltpu.sync_copy(zb, ids_hbm.at[pl.ds(E, FC)])
            pltpu.sync_copy(zb, dsts_hbm.at[pl.ds(E, FC)])

    return k(pos, dst)


def _k2(u, v, src, dst):
    mesh = plsc.VectorSubcoreMesh(core_axis_name="c", subcore_axis_name="s")

    @functools.partial(
        pl.kernel,
        out_type=(jax.ShapeDtypeStruct((E, D), jnp.float32),
                  jax.ShapeDtypeStruct((NW, 2, D), jnp.float32)),
        mesh=mesh,
        scratch_types=[
            pltpu.VMEM((GC,), jnp.int32),
            pltpu.VMEM((GC,), jnp.int32),
            pltpu.VMEM((GC, D), jnp.float32),
            pltpu.VMEM((GC, D), jnp.float32),
            pltpu.VMEM((2, D), jnp.float32),
            pltpu.SemaphoreType.DMA,
            pltpu.SemaphoreType.DMA,
        ],
    )
    def k(u_hbm, v_hbm, src_hbm, dst_hbm, z_hbm, st_hbm,
          idx_d, idx_s, ub, vb, stv, sem1, sem2):
        c = lax.axis_index("c")
        s = lax.axis_index("s")
        wid = s * 2 + c
        base = wid * EPW

        def chunk(j, acc):
            su, sq = acc
            row0 = base + j * GC
            pltpu.sync_copy(dst_hbm.at[pl.ds(row0, GC)], idx_d)
            pltpu.sync_copy(src_hbm.at[pl.ds(row0, GC)], idx_s)
            cps = []
            for k_ in range(GC // SB):
                sl = pl.ds(k_ * SB, SB)
                cps.append(pltpu.async_copy(
                    u_hbm.at[idx_d.at[sl]], ub.at[sl], sem1))
                cps.append(pltpu.async_copy(
                    v_hbm.at[idx_s.at[sl]], vb.at[sl], sem2))
            for cp in cps:
                cp.wait()

            def row(r, acc2):
                su2, sq2 = acc2
                nsu = []
                nsq = []
                for kk in range(8):
                    sl = pl.ds(kk * 16, 16)
                    z = ub[r, sl] + vb[r, sl]
                    ub[r, sl] = z
                    nsu.append(su2[kk] + z)
                    nsq.append(sq2[kk] + z * z)
                return tuple(nsu), tuple(nsq)

            acc = lax.fori_loop(0, GC, row, (su, sq))
            pltpu.sync_copy(ub, z_hbm.at[pl.ds(row0, GC)])
            return acc

        zero = tuple(jnp.zeros((16,), jnp.float32) for _ in range(8))
        su, sq = lax.fori_loop(0, EPW // GC, chunk, (zero, zero))
        for kk in range(8):
            sl = pl.ds(kk * 16, 16)
            stv[0, sl] = su[kk]
            stv[1, sl] = sq[kk]
        pltpu.sync_copy(stv, st_hbm.at[wid])

    return k(u, v, src, dst)


def _mlp1_body(st1_ref, z_ref, g1_ref, be1_ref, w2_ref, z2_ref, st2_ref,
               acc_ref):
    t = pl.program_id(0)

    @pl.when(t == 0)
    def _init():
        acc_ref[...] = jnp.zeros_like(acc_ref)

    st = jnp.sum(st1_ref[...], axis=0)
    mu = st[0:1] / E
    var = st[1:2] / E - mu * mu
    inv = lax.rsqrt(var + EPS)
    z1 = z_ref[...]
    h1 = (z1 - mu) * (inv * g1_ref[...]) + be1_ref[...]
    h1 = jnp.where(h1 >= 0, h1, 0.01 * h1)
    z2 = jnp.dot(h1, w2_ref[...], preferred_element_type=jnp.float32)
    z2_ref[...] = z2
    acc_ref[0:1] += jnp.sum(z2, axis=0, keepdims=True)
    acc_ref[1:2] += jnp.sum(z2 * z2, axis=0, keepdims=True)

    @pl.when(t == T - 1)
    def _emit():
        st2_ref[...] = acc_ref[...]


def _k34(st1, z1, g1, be1, w2t):
    vec = pl.BlockSpec((1, D), lambda t: (0, 0))
    return pl.pallas_call(
        _mlp1_body,
        out_shape=(jax.ShapeDtypeStruct((E, D), jnp.float32),
                   jax.ShapeDtypeStruct((8, D), jnp.float32)),
        grid=(T,),
        in_specs=[
            pl.BlockSpec((NW, 2, D), lambda t: (0, 0, 0)),
            pl.BlockSpec((BT, D), lambda t: (t, 0)),
            vec, vec,
            pl.BlockSpec((D, D), lambda t: (0, 0)),
        ],
        out_specs=(
            pl.BlockSpec((BT, D), lambda t: (t, 0)),
            pl.BlockSpec((8, D), lambda t: (0, 0)),
        ),
        scratch_shapes=[pltpu.VMEM((8, D), jnp.float32)],
    )(st1, z1, g1.reshape(1, D), be1.reshape(1, D), w2t)


def _mlp2_body(st2_ref, z_ref, g2_ref, be2_ref, w3_ref, h3_ref):
    mu = st2_ref[0:1] / E
    var = st2_ref[1:2] / E - mu * mu
    inv = lax.rsqrt(var + EPS)
    z2 = z_ref[...]
    h2 = (z2 - mu) * (inv * g2_ref[...]) + be2_ref[...]
    h2 = jnp.where(h2 >= 0, h2, 0.01 * h2)
    h3_ref[...] = jnp.dot(h2, w3_ref[...], preferred_element_type=jnp.float32)


def _k5(st2, z2, g2, be2, w3t):
    vec = pl.BlockSpec((1, D), lambda t: (0, 0))
    return pl.pallas_call(
        _mlp2_body,
        out_shape=jax.ShapeDtypeStruct((E, D), jnp.float32),
        grid=(T,),
        in_specs=[
            pl.BlockSpec((8, D), lambda t: (0, 0)),
            pl.BlockSpec((BT, D), lambda t: (t, 0)),
            vec, vec,
            pl.BlockSpec((D, D), lambda t: (0, 0)),
        ],
        out_specs=pl.BlockSpec((BT, D), lambda t: (t, 0)),
    )(st2, z2, g2.reshape(1, D), be2.reshape(1, D), w3t)


def _k6(ids_s, dst_s, off, h3):
    mesh = plsc.VectorSubcoreMesh(core_axis_name="c", subcore_axis_name="s")

    @functools.partial(
        pl.kernel,
        out_type=jax.ShapeDtypeStruct((NPAD, D), jnp.float32),
        mesh=mesh,
        scratch_types=[
            pltpu.VMEM((NPW, D), jnp.float32),
            pltpu.VMEM((FC,), jnp.int32),
            pltpu.VMEM((FC,), jnp.int32),
            pltpu.VMEM((FC, D), jnp.float32),
            pltpu.VMEM((16,), jnp.int32),
            pltpu.SemaphoreType.DMA,
        ],
    )
    def k(off_hbm, ids_hbm, dsts_hbm, h3_hbm, agg_hbm,
          acc, idsb, db, rows, offv, sem1):
        c = lax.axis_index("c")
        s = lax.axis_index("s")
        wid = s * 2 + c
        lo = wid * NPW
        hi = lo + NPW
        neg = jnp.full((16,), -jnp.inf, jnp.float32)

        pltpu.sync_copy(off_hbm.at[wid], offv)
        ovec = offv[pl.ds(0, 16)]
        start = ovec[0]
        end = ovec[1]

        def initacc(i, _):
            for kk in range(8):
                acc[i, pl.ds(kk * 16, 16)] = neg
            return 0
        lax.fori_loop(0, NPW, initacc, 0)

        start_al = jnp.clip(start - (start & 7), 0, E)
        nch = jnp.clip((end - start_al + FC - 1) // FC, 0, E // FC + 1)

        def chunk(j, _):
            o = pl.multiple_of(jnp.clip(start_al + j * FC, 0, E), 8)
            pltpu.sync_copy(ids_hbm.at[pl.ds(o, FC)], idsb)
            cp1 = pltpu.async_copy(h3_hbm.at[idsb.at[pl.ds(0, 128)]],
                                   rows.at[pl.ds(0, 128)], sem1)
            cp2 = pltpu.async_copy(h3_hbm.at[idsb.at[pl.ds(128, 128)]],
                                   rows.at[pl.ds(128, 128)], sem1)
            pltpu.sync_copy(dsts_hbm.at[pl.ds(o, FC)], db)
            cp1.wait()
            cp2.wait()

            def grp(g, _):
                gof = pl.multiple_of(g * 16, 16)
                dv = db[pl.ds(gof, 16)]
                for j in range(16):
                    d = dv[j]
                    ok = jnp.logical_and(d >= lo, d < hi)

                    @pl.when(ok)
                    def _apply(d=d, j=j):
                        r = d - lo
                        i = gof + j
                        for kk in range(8):
                            sl = pl.ds(kk * 16, 16)
                            acc[r, sl] = jnp.maximum(acc[r, sl], rows[i, sl])
                return 0
            lax.fori_loop(0, FC // 16, grp, 0)
            return 0
        lax.fori_loop(0, nch, chunk, 0)

        pltpu.sync_copy(acc, agg_hbm.at[pl.ds(lo, NPW)])

    return k(off, ids_s, dst_s, h3)


def _final_body(agg_ref, x_ref, b3_ref, o_ref):
    a = agg_ref[...]
    a = jnp.where(jnp.isneginf(a), 0.0, a + b3_ref[...])
    r = a + x_ref[...]
    o_ref[...] = jnp.where(r >= 0, r, 0.01 * r)


def _k7(agg, x, b3):
    return pl.pallas_call(
        _final_body,
        out_shape=jax.ShapeDtypeStruct((N, D), jnp.float32),
        grid=(10,),
        in_specs=[
            pl.BlockSpec((N // 10, D), lambda i: (i, 0)),
            pl.BlockSpec((N // 10, D), lambda i: (i, 0)),
            pl.BlockSpec((1, D), lambda i: (0, 0)),
        ],
        out_specs=pl.BlockSpec((N // 10, D), lambda i: (i, 0)),
    )(agg, x, b3.reshape(1, D))


def kernel(x, edge_index, W1, b1, g1, be1, W2, b2, g2, be2, W3, b3):
    src = edge_index[0]
    dst = edge_index[1]
    wa = (W1[:, :D] - W1[:, D:]).T
    wb = W1[:, D:].T

    pos3, cnt = _kpos(dst.reshape(TP, 1, BP))
    cnts = cnt[0].astype(jnp.int32)
    ends = jnp.cumsum(cnts)
    starts = ends - cnts
    off = jnp.stack([starts, ends], axis=1)
    off = jnp.pad(off, ((0, 0), (0, 14)))
    ids_s, dst_s = _kperm(pos3.reshape(E), dst)

    u, v = _k1(x, wa, wb)
    z1, st1 = _k2(u, v, src, dst)
    z2, st2 = _k34(st1, z1, g1, be1, W2.T)
    h3 = _k5(st2, z2, g2, be2, W3.T)
    agg = _k6(ids_s, dst_s, off, h3)
    return _k7(agg[:N], x, b3)


# speedup vs baseline: 1.3353x; 1.0021x over previous
import functools

import jax
import jax.numpy as jnp
from jax import lax
from jax.experimental import pallas as pl
from jax.experimental.pallas import tpu as pltpu
from jax.experimental.pallas import tpu_sc as plsc

EPS = 1e-5
N = 10000
E = 320000
D = 128
NW = 32
EPW = E // NW
GC = 400
NPW = 320
NPAD = NW * NPW
BP = 1280
TP = E // BP
PC = 400
SB = 80
NSB = PC // SB
FC = 256
BT = 1280
T = E // BT


def _uv_body(x_ref, wa_ref, wb_ref, u_ref, v_ref):
    xb = x_ref[...]
    u_ref[...] = jnp.dot(xb, wa_ref[...], preferred_element_type=jnp.float32)
    v_ref[...] = jnp.dot(xb, wb_ref[...], preferred_element_type=jnp.float32)


def _k1(x, wa, wb):
    return pl.pallas_call(
        _uv_body,
        out_shape=(jax.ShapeDtypeStruct((N, D), jnp.float32),
                   jax.ShapeDtypeStruct((N, D), jnp.float32)),
        grid=(10,),
        in_specs=[
            pl.BlockSpec((N // 10, D), lambda i: (i, 0)),
            pl.BlockSpec((D, D), lambda i: (0, 0)),
            pl.BlockSpec((D, D), lambda i: (0, 0)),
        ],
        out_specs=(pl.BlockSpec((N // 10, D), lambda i: (i, 0)),
                   pl.BlockSpec((N // 10, D), lambda i: (i, 0))),
    )(x, wa, wb)


def _pos_body(dst_ref, pos_ref, cnt_ref, acc_ref, l_ref):
    p = pl.program_id(0)
    t = pl.program_id(1)

    @pl.when(jnp.logical_and(p == 0, t == 0))
    def _init():
        acc_ref[...] = jnp.zeros_like(acc_ref)
        ri = lax.broadcasted_iota(jnp.int32, (BP, BP), 0)
        ci = lax.broadcasted_iota(jnp.int32, (BP, BP), 1)
        l_ref[...] = jnp.where(ri > ci, 1.0, 0.0)

    d = dst_ref[0]
    bkt = d.reshape(BP, 1) // NPW
    bi = lax.broadcasted_iota(jnp.int32, (BP, NW), 1)
    oh = jnp.where(bkt == bi, 1.0, 0.0)

    @pl.when(p == 0)
    def _count():
        acc_ref[0:1] += jnp.sum(oh, axis=0, keepdims=True)

    @pl.when(jnp.logical_and(p == 1, t == 0))
    def _mkstarts():
        cnt_ref[...] = acc_ref[0:1]
        cs = acc_ref[0:1]
        for sh in (1, 2, 4, 8, 16):
            cs = cs + jnp.concatenate(
                [jnp.zeros((1, sh), jnp.float32), cs[:, :-sh]], axis=1)
        acc_ref[2:3] = cs - acc_ref[0:1]
        acc_ref[0:1] = jnp.zeros((1, NW), jnp.float32)

    @pl.when(p == 1)
    def _emitpos():
        pw = jnp.dot(l_ref[...], oh, preferred_element_type=jnp.float32)
        base = acc_ref[0:1] + acc_ref[2:3]
        pp = jnp.sum(oh * (pw + base), axis=1)
        pos_ref[0] = pp.astype(jnp.int32).reshape(1, BP)
        acc_ref[0:1] += jnp.sum(oh, axis=0, keepdims=True)


def _kpos(dst3):
    pos3, cnt = pl.pallas_call(
        _pos_body,
        out_shape=(jax.ShapeDtypeStruct((TP, 1, BP), jnp.int32),
                   jax.ShapeDtypeStruct((1, NW), jnp.float32)),
        grid=(2, TP),
        in_specs=[pl.BlockSpec((1, 1, BP), lambda p, t: (t, 0, 0))],
        out_specs=(pl.BlockSpec((1, 1, BP),
                                lambda p, t: (jnp.where(p == 1, t, 0), 0, 0)),
                   pl.BlockSpec((1, NW), lambda p, t: (0, 0))),
        scratch_shapes=[pltpu.VMEM((8, NW), jnp.float32),
                        pltpu.VMEM((BP, BP), jnp.float32)],
    )(dst3)
    return pos3, cnt


def _kperm(pos, dst):
    mesh = plsc.VectorSubcoreMesh(core_axis_name="c", subcore_axis_name="s")

    @functools.partial(
        pl.kernel,
        out_type=(jax.ShapeDtypeStruct((E + FC,), jnp.int32),
                  jax.ShapeDtypeStruct((E + FC,), jnp.int32)),
        mesh=mesh,
        scratch_types=[
            pltpu.VMEM((NSB, SB), jnp.int32),
            pltpu.VMEM((NSB, SB), jnp.int32),
            pltpu.VMEM((NSB, SB), jnp.int32),
            pltpu.VMEM((FC,), jnp.int32),
            pltpu.SemaphoreType.DMA,
            pltpu.SemaphoreType.DMA,
        ],
    )
    def k(pos_hbm, dst_hbm, ids_hbm, dsts_hbm, posb, db, idb, zb, sem1, sem2):
        c = lax.axis_index("c")
        s = lax.axis_index("s")
        wid = s * 2 + c
        base = wid * EPW
        iota = lax.iota(jnp.int32, 16)

        def chunk(j, _):
            e0 = base + j * PC
            for k_ in range(NSB):
                pltpu.sync_copy(pos_hbm.at[pl.ds(e0 + k_ * SB, SB)],
                                posb.at[k_])
                pltpu.sync_copy(dst_hbm.at[pl.ds(e0 + k_ * SB, SB)],
                                db.at[k_])

            def mk_ids(i, _):
                for k_ in range(NSB):
                    sl = pl.ds(pl.multiple_of(i * 16, 16), 16)
                    idb[k_, sl] = (jnp.full((16,), e0 + k_ * SB, jnp.int32)
                                   + i * 16 + iota)
                return 0
            lax.fori_loop(0, SB // 16, mk_ids, 0)
            for k_ in range(NSB):
                cp1 = pltpu.async_copy(idb.at[k_], ids_hbm.at[posb.at[k_]],
                                       sem1)
                cp2 = pltpu.async_copy(db.at[k_], dsts_hbm.at[posb.at[k_]],
                                       sem2)
                cp1.wait()
                cp2.wait()
            return 0
        lax.fori_loop(0, EPW // PC, chunk, 0)

        @pl.when(wid == NW - 1)
        def _pad():
            def z(i, _):
                zb[pl.ds(pl.multiple_of(i * 16, 16), 16)] = jnp.zeros(
                    (16,), jnp.int32)
                return 0
            lax.fori_loop(0, FC // 16, z, 0)
            pltpu.sync_copy(zb, ids_hbm.at[pl.ds(E, FC)])
            pltpu.sync_copy(zb, dsts_hbm.at[pl.ds(E, FC)])

    return k(pos, dst)


def _k2(u, v, src, dst):
    mesh = plsc.VectorSubcoreMesh(core_axis_name="c", subcore_axis_name="s")

    @functools.partial(
        pl.kernel,
        out_type=(jax.ShapeDtypeStruct((E, D), jnp.float32),
                  jax.ShapeDtypeStruct((NW, 2, D), jnp.float32)),
        mesh=mesh,
        scratch_types=[
            pltpu.VMEM((GC,), jnp.int32),
            pltpu.VMEM((GC,), jnp.int32),
            pltpu.VMEM((GC, D), jnp.float32),
            pltpu.VMEM((GC, D), jnp.float32),
            pltpu.VMEM((2, D), jnp.float32),
            pltpu.SemaphoreType.DMA,
            pltpu.SemaphoreType.DMA,
        ],
    )
    def k(u_hbm, v_hbm, src_hbm, dst_hbm, z_hbm, st_hbm,
          idx_d, idx_s, ub, vb, stv, sem1, sem2):
        c = lax.axis_index("c")
        s = lax.axis_index("s")
        wid = s * 2 + c
        base = wid * EPW

        def chunk(j, acc):
            su, sq = acc
            row0 = base + j * GC
            pltpu.sync_copy(dst_hbm.at[pl.ds(row0, GC)], idx_d)
            pltpu.sync_copy(src_hbm.at[pl.ds(row0, GC)], idx_s)
            cps = []
            for k_ in range(GC // SB):
                sl = pl.ds(k_ * SB, SB)
                cps.append(pltpu.async_copy(
                    u_hbm.at[idx_d.at[sl]], ub.at[sl], sem1))
                cps.append(pltpu.async_copy(
                    v_hbm.at[idx_s.at[sl]], vb.at[sl], sem2))
            for cp in cps:
                cp.wait()

            def row(r, acc2):
                su2, sq2 = acc2
                nsu = []
                nsq = []
                for kk in range(8):
                    sl = pl.ds(kk * 16, 16)
                    z = ub[r, sl] + vb[r, sl]
                    ub[r, sl] = z
                    nsu.append(su2[kk] + z)
                    nsq.append(sq2[kk] + z * z)
                return tuple(nsu), tuple(nsq)

            acc = lax.fori_loop(0, GC, row, (su, sq))
            pltpu.sync_copy(ub, z_hbm.at[pl.ds(row0, GC)])
            return acc

        zero = tuple(jnp.zeros((16,), jnp.float32) for _ in range(8))
        su, sq = lax.fori_loop(0, EPW // GC, chunk, (zero, zero))
        for kk in range(8):
            sl = pl.ds(kk * 16, 16)
            stv[0, sl] = su[kk]
            stv[1, sl] = sq[kk]
        pltpu.sync_copy(stv, st_hbm.at[wid])

    return k(u, v, src, dst)


def _mlp1_body(st1_ref, z_ref, g1_ref, be1_ref, w2_ref, z2_ref, st2_ref,
               acc_ref):
    t = pl.program_id(0)

    @pl.when(t == 0)
    def _init():
        acc_ref[...] = jnp.zeros_like(acc_ref)

    st = jnp.sum(st1_ref[...], axis=0)
    mu = st[0:1] / E
    var = st[1:2] / E - mu * mu
    inv = lax.rsqrt(var + EPS)
    z1 = z_ref[...]
    h1 = (z1 - mu) * (inv * g1_ref[...]) + be1_ref[...]
    h1 = jnp.where(h1 >= 0, h1, 0.01 * h1)
    z2 = jnp.dot(h1, w2_ref[...], preferred_element_type=jnp.float32)
    z2_ref[...] = z2
    acc_ref[0:1] += jnp.sum(z2, axis=0, keepdims=True)
    acc_ref[1:2] += jnp.sum(z2 * z2, axis=0, keepdims=True)

    @pl.when(t == T - 1)
    def _emit():
        st2_ref[...] = acc_ref[...]


def _k34(st1, z1, g1, be1, w2t):
    vec = pl.BlockSpec((1, D), lambda t: (0, 0))
    return pl.pallas_call(
        _mlp1_body,
        out_shape=(jax.ShapeDtypeStruct((E, D), jnp.float32),
                   jax.ShapeDtypeStruct((8, D), jnp.float32)),
        grid=(T,),
        in_specs=[
            pl.BlockSpec((NW, 2, D), lambda t: (0, 0, 0)),
            pl.BlockSpec((BT, D), lambda t: (t, 0)),
            vec, vec,
            pl.BlockSpec((D, D), lambda t: (0, 0)),
        ],
        out_specs=(
            pl.BlockSpec((BT, D), lambda t: (t, 0)),
            pl.BlockSpec((8, D), lambda t: (0, 0)),
        ),
        scratch_shapes=[pltpu.VMEM((8, D), jnp.float32)],
    )(st1, z1, g1.reshape(1, D), be1.reshape(1, D), w2t)


def _mlp2_body(st2_ref, z_ref, g2_ref, be2_ref, w3_ref, h3_ref):
    mu = st2_ref[0:1] / E
    var = st2_ref[1:2] / E - mu * mu
    inv = lax.rsqrt(var + EPS)
    z2 = z_ref[...]
    h2 = (z2 - mu) * (inv * g2_ref[...]) + be2_ref[...]
    h2 = jnp.where(h2 >= 0, h2, 0.01 * h2)
    h3_ref[...] = jnp.dot(h2, w3_ref[...], preferred_element_type=jnp.float32)


def _k5(st2, z2, g2, be2, w3t):
    vec = pl.BlockSpec((1, D), lambda t: (0, 0))
    return pl.pallas_call(
        _mlp2_body,
        out_shape=jax.ShapeDtypeStruct((E, D), jnp.float32),
        grid=(T,),
        in_specs=[
            pl.BlockSpec((8, D), lambda t: (0, 0)),
            pl.BlockSpec((BT, D), lambda t: (t, 0)),
            vec, vec,
            pl.BlockSpec((D, D), lambda t: (0, 0)),
        ],
        out_specs=pl.BlockSpec((BT, D), lambda t: (t, 0)),
    )(st2, z2, g2.reshape(1, D), be2.reshape(1, D), w3t)


def _k6(ids_s, dst_s, off, h3):
    mesh = plsc.VectorSubcoreMesh(core_axis_name="c", subcore_axis_name="s")

    @functools.partial(
        pl.kernel,
        out_type=jax.ShapeDtypeStruct((NPAD, D), jnp.float32),
        mesh=mesh,
        scratch_types=[
            pltpu.VMEM((NPW, D), jnp.float32),
            pltpu.VMEM((FC,), jnp.int32),
            pltpu.VMEM((FC,), jnp.int32),
            pltpu.VMEM((FC, D), jnp.float32),
            pltpu.VMEM((16,), jnp.int32),
            pltpu.SemaphoreType.DMA,
        ],
    )
    def k(off_hbm, ids_hbm, dsts_hbm, h3_hbm, agg_hbm,
          acc, idsb, db, rows, offv, sem1):
        c = lax.axis_index("c")
        s = lax.axis_index("s")
        wid = s * 2 + c
        lo = wid * NPW
        hi = lo + NPW
        neg = jnp.full((16,), -jnp.inf, jnp.float32)

        pltpu.sync_copy(off_hbm.at[wid], offv)
        ovec = offv[pl.ds(0, 16)]
        start = ovec[0]
        end = ovec[1]

        def initacc(i, _):
            for kk in range(8):
                acc[i, pl.ds(kk * 16, 16)] = neg
            return 0
        lax.fori_loop(0, NPW, initacc, 0)

        start_al = jnp.clip(start - (start & 7), 0, E)
        nch = jnp.clip((end - start_al + FC - 1) // FC, 0, E // FC + 1)

        def chunk(j, _):
            o = pl.multiple_of(jnp.clip(start_al + j * FC, 0, E), 8)
            pltpu.sync_copy(ids_hbm.at[pl.ds(o, FC)], idsb)
            cp1 = pltpu.async_copy(h3_hbm.at[idsb.at[pl.ds(0, 128)]],
                                   rows.at[pl.ds(0, 128)], sem1)
            cp2 = pltpu.async_copy(h3_hbm.at[idsb.at[pl.ds(128, 128)]],
                                   rows.at[pl.ds(128, 128)], sem1)
            pltpu.sync_copy(dsts_hbm.at[pl.ds(o, FC)], db)
            cp1.wait()
            cp2.wait()

            def grp(g, _):
                gof = pl.multiple_of(g * 16, 16)
                dv = db[pl.ds(gof, 16)]
                for j in range(16):
                    d = dv[j]
                    ok = jnp.logical_and(d >= lo, d < hi)

                    @pl.when(ok)
                    def _apply(d=d, j=j):
                        r = d - lo
                        i = gof + j
                        for kk in range(8):
                            sl = pl.ds(kk * 16, 16)
                            acc[r, sl] = jnp.maximum(acc[r, sl], rows[i, sl])
                return 0
            lax.fori_loop(0, FC // 16, grp, 0)
            return 0
        lax.fori_loop(0, nch, chunk, 0)

        pltpu.sync_copy(acc, agg_hbm.at[pl.ds(lo, NPW)])

    return k(off, ids_s, dst_s, h3)


def _final_body(agg_ref, x_ref, b3_ref, o_ref):
    a = agg_ref[...]
    a = jnp.where(jnp.isneginf(a), 0.0, a + b3_ref[...])
    r = a + x_ref[...]
    o_ref[...] = jnp.where(r >= 0, r, 0.01 * r)


def _k7(agg, x, b3):
    return pl.pallas_call(
        _final_body,
        out_shape=jax.ShapeDtypeStruct((N, D), jnp.float32),
        grid=(10,),
        in_specs=[
            pl.BlockSpec((N // 10, D), lambda i: (i, 0)),
            pl.BlockSpec((N // 10, D), lambda i: (i, 0)),
            pl.BlockSpec((1, D), lambda i: (0, 0)),
        ],
        out_specs=pl.BlockSpec((N // 10, D), lambda i: (i, 0)),
    )(agg, x, b3.reshape(1, D))


def kernel(x, edge_index, W1, b1, g1, be1, W2, b2, g2, be2, W3, b3):
    src = edge_index[0]
    dst = edge_index[1]
    wa = (W1[:, :D] - W1[:, D:]).T
    wb = W1[:, D:].T

    pos3, cnt = _kpos(dst.reshape(TP, 1, BP))
    cnts = cnt[0].astype(jnp.int32)
    ends = jnp.cumsum(cnts)
    starts = ends - cnts
    off = jnp.stack([starts, ends], axis=1)
    off = jnp.pad(off, ((0, 0), (0, 14)))
    ids_s, dst_s = _kperm(pos3.reshape(E), dst)

    u, v = _k1(x, wa, wb)
    z1, st1 = _k2(u, v, src, dst)
    z2, st2 = _k34(st1, z1, g1, be1, W2.T)
    h3 = _k5(st2, z2, g2, be2, W3.T)
    agg = _k6(ids_s, dst_s, off, h3)
    return _k7(agg[:N], x, b3)
